# baseline (speedup 1.0000x reference)
import jax
import jax.numpy as jnp
from jax import lax
from jax.experimental import pallas as pl
from jax.experimental.pallas import tpu as pltpu
from jax.experimental.pallas import tpu_sc as plsc

VOCAB = 100000
EMBED = 64
BATCH = 4096
HIST = 50

NC = 2
NS = 16
LANES = 16
NJ = EMBED // LANES
NW = NC * NS
B_W = BATCH // NW
SPU = 2
IPU = SPU * HIST
U_W = B_W // SPU
NBUF = 4

VPAD = 100096
NCHUNK = VPAD // 128
CPT = 25


def _transpose_body(tabT_hbm, lin_hbm, in_v, out_v, *sems):
    wid = lax.axis_index("s") * NC + lax.axis_index("c")
    in_sems, out_sems = sems[:2], sems[2:]

    def fire_in(c, b):
        return pltpu.async_copy(tabT_hbm.at[:, pl.ds(c * 128, 128)],
                                in_v.at[b], in_sems[b])

    def wait_in(c, b):
        pltpu.make_async_copy(tabT_hbm.at[:, pl.ds(c * 128, 128)],
                              in_v.at[b], in_sems[b]).wait()

    def fire_out(c, b):
        return pltpu.async_copy(out_v.at[b], lin_hbm.at[pl.ds(c * 8192, 8192)],
                                out_sems[b])

    def wait_out(c, b):
        pltpu.make_async_copy(out_v.at[b], lin_hbm.at[pl.ds(c * 8192, 8192)],
                              out_sems[b]).wait()

    rowi = [lax.iota(jnp.int32, LANES) + 16 * g for g in range(NJ)]

    @pl.when(wid < NCHUNK)
    def _():
        fire_in(wid, 0)

    for k in range(CPT):
        b = k % 2
        c = wid + 32 * k

        @pl.when(c < NCHUNK)
        def _():
            wait_in(c, b)
            @pl.when(c + 32 < NCHUNK)
            def _():
                fire_in(c + 32, 1 - b)
            if k >= 2:
                wait_out(c - 64, b)

            def col(j, carry):
                colj = jnp.full((LANES,), j, jnp.int32)
                for g in range(NJ):
                    v = plsc.load_gather(in_v.at[b], [rowi[g], colj])
                    out_v[b, pl.ds(j * EMBED + g * LANES, LANES)] = v
                return carry

            lax.fori_loop(0, 128, col, 0)
            fire_out(c, b)

    for k in range(CPT - 3, CPT):
        b = k % 2
        c = wid + 32 * k

        @pl.when((c < NCHUNK) & (c + 64 >= NCHUNK))
        def _():
            wait_out(c, b)


def _gather_body(idx_hbm, table_hbm, out_hbm, idx_v, rows_v, out_v, *sems):
    wid = lax.axis_index("s") * NC + lax.axis_index("c")
    ubase = wid * U_W

    pltpu.sync_copy(idx_hbm.at[pl.ds(ubase, U_W)], idx_v)

    def fire(u, b):
        return pltpu.async_copy(table_hbm.at[idx_v.at[u]], rows_v.at[b], sems[b])

    for b in range(NBUF):
        fire(b, b)

    def group(gi, carry):
        for b in range(NBUF):
            u = gi * NBUF + b
            pltpu.make_async_copy(table_hbm.at[idx_v.at[u]], rows_v.at[b],
                                  sems[b]).wait()
            for p in range(SPU):
                base_t = p * HIST

                def tok(i, accs):
                    t = 2 * i
                    return tuple(
                        accs[k * NJ + j]
                        + rows_v[b, base_t + t + k, pl.ds(j * LANES, LANES)]
                        for k in range(2) for j in range(NJ)
                    )

                zero = jnp.zeros((LANES,), jnp.float32)
                accs = lax.fori_loop(0, HIST // 2, tok, (zero,) * (2 * NJ))
                s = SPU * u + p
                for j in range(NJ):
                    out_v[s, pl.ds(j * LANES, LANES)] = (
                        (accs[j] + accs[NJ + j]) * (1.0 / HIST))
            @pl.when(u + NBUF < U_W)
            def _():
                fire(u + NBUF, b)
        return carry

    lax.fori_loop(0, U_W // NBUF, group, 0)

    pltpu.sync_copy(out_v, out_hbm.at[pl.ds(wid * B_W, B_W)])


@jax.jit
def _emb_mean(idx, table):
    mesh = plsc.VectorSubcoreMesh(core_axis_name="c", subcore_axis_name="s")

    lin = pl.kernel(
        _transpose_body,
        out_type=jax.ShapeDtypeStruct((VPAD * EMBED,), jnp.float32),
        mesh=mesh,
        compiler_params=pltpu.CompilerParams(use_tc_tiling_on_sc=True,
                                             needs_layout_passes=False),
        scratch_types=[
            pltpu.VMEM((2, EMBED, 128), jnp.float32),
            pltpu.VMEM((2, 128 * EMBED), jnp.float32),
        ] + [pltpu.SemaphoreType.DMA] * 4,
    )(table.T)

    table_lin = lin.reshape(VPAD, EMBED)

    return pl.kernel(
        _gather_body,
        out_type=jax.ShapeDtypeStruct((BATCH, EMBED), jnp.float32),
        mesh=mesh,
        compiler_params=pltpu.CompilerParams(use_tc_tiling_on_sc=False,
                                             needs_layout_passes=False),
        scratch_types=[
            pltpu.VMEM((U_W, IPU), jnp.int32),
            pltpu.VMEM((NBUF, IPU, EMBED), jnp.float32),
            pltpu.VMEM((B_W, EMBED), jnp.float32),
        ] + [pltpu.SemaphoreType.DMA] * NBUF,
    )(idx, table_lin)


def kernel(input_var, table):
    idx = input_var.astype(jnp.int32).reshape(BATCH // SPU, IPU)
    return _emb_mean(idx, table)


# speedup vs baseline: 1.4894x; 1.4894x over previous
import jax
import jax.numpy as jnp
from jax import lax
from jax.experimental import pallas as pl
from jax.experimental.pallas import tpu as pltpu
from jax.experimental.pallas import tpu_sc as plsc

VOCAB = 100000
EMBED = 64
BATCH = 4096
HIST = 50

NC = 2
NS = 16
LANES = 16
NJ = EMBED // LANES
NW = NC * NS
B_W = BATCH // NW
SPU = 2
IPU = SPU * HIST
U_W = B_W // SPU
NBUF = 4

VPAD = 100096
NCHUNK = VPAD // 128
CPT = 25


def _transpose_body(tabT_hbm, lin_hbm, in_v, out_v, *sems):
    wid = lax.axis_index("s") * NC + lax.axis_index("c")
    in_sems, out_sems = sems[:2], sems[2:]

    def fire_in(c, b):
        return pltpu.async_copy(tabT_hbm.at[:, pl.ds(c * 128, 128)],
                                in_v.at[b], in_sems[b])

    def wait_in(c, b):
        pltpu.make_async_copy(tabT_hbm.at[:, pl.ds(c * 128, 128)],
                              in_v.at[b], in_sems[b]).wait()

    def fire_out(c, b):
        return pltpu.async_copy(out_v.at[b], lin_hbm.at[pl.ds(c * 8192, 8192)],
                                out_sems[b])

    def wait_out(c, b):
        pltpu.make_async_copy(out_v.at[b], lin_hbm.at[pl.ds(c * 8192, 8192)],
                              out_sems[b]).wait()

    rowi = [lax.iota(jnp.int32, LANES) + 16 * g for g in range(NJ)]

    @pl.when(wid < NCHUNK)
    def _():
        fire_in(wid, 0)

    for k in range(CPT):
        b = k % 2
        c = wid + 32 * k

        @pl.when(c < NCHUNK)
        def _():
            wait_in(c, b)
            @pl.when(c + 32 < NCHUNK)
            def _():
                fire_in(c + 32, 1 - b)
            if k >= 2:
                wait_out(c - 64, b)

            @plsc.parallel_loop(0, 128, 1, unroll=4)
            def _(j):
                colj = jnp.full((LANES,), j, jnp.int32)
                for g in range(NJ):
                    v = plsc.load_gather(in_v.at[b], [rowi[g], colj])
                    out_v[b, pl.ds(j * EMBED + g * LANES, LANES)] = v

            fire_out(c, b)

    for k in range(CPT - 3, CPT):
        b = k % 2
        c = wid + 32 * k

        @pl.when((c < NCHUNK) & (c + 64 >= NCHUNK))
        def _():
            wait_out(c, b)


def _gather_body(idx_hbm, table_hbm, out_hbm, idx_v, rows_v, out_v, *sems):
    wid = lax.axis_index("s") * NC + lax.axis_index("c")
    ubase = wid * U_W

    pltpu.sync_copy(idx_hbm.at[pl.ds(ubase, U_W)], idx_v)

    def fire(u, b):
        return pltpu.async_copy(table_hbm.at[idx_v.at[u]], rows_v.at[b], sems[b])

    for b in range(NBUF):
        fire(b, b)

    def group(gi, carry):
        for b in range(NBUF):
            u = gi * NBUF + b
            pltpu.make_async_copy(table_hbm.at[idx_v.at[u]], rows_v.at[b],
                                  sems[b]).wait()
            for p in range(SPU):
                base_t = p * HIST

                def tok(i, accs):
                    t = 2 * i
                    return tuple(
                        accs[k * NJ + j]
                        + rows_v[b, base_t + t + k, pl.ds(j * LANES, LANES)]
                        for k in range(2) for j in range(NJ)
                    )

                zero = jnp.zeros((LANES,), jnp.float32)
                accs = lax.fori_loop(0, HIST // 2, tok, (zero,) * (2 * NJ))
                s = SPU * u + p
                for j in range(NJ):
                    out_v[s, pl.ds(j * LANES, LANES)] = (
                        (accs[j] + accs[NJ + j]) * (1.0 / HIST))
            @pl.when(u + NBUF < U_W)
            def _():
                fire(u + NBUF, b)
        return carry

    lax.fori_loop(0, U_W // NBUF, group, 0)

    pltpu.sync_copy(out_v, out_hbm.at[pl.ds(wid * B_W, B_W)])


@jax.jit
def _emb_mean(idx, table):
    mesh = plsc.VectorSubcoreMesh(core_axis_name="c", subcore_axis_name="s")

    lin = pl.kernel(
        _transpose_body,
        out_type=jax.ShapeDtypeStruct((VPAD * EMBED,), jnp.float32),
        mesh=mesh,
        compiler_params=pltpu.CompilerParams(use_tc_tiling_on_sc=True,
                                             needs_layout_passes=False),
        scratch_types=[
            pltpu.VMEM((2, EMBED, 128), jnp.float32),
            pltpu.VMEM((2, 128 * EMBED), jnp.float32),
        ] + [pltpu.SemaphoreType.DMA] * 4,
    )(table.T)

    table_lin = lin.reshape(VPAD, EMBED)

    return pl.kernel(
        _gather_body,
        out_type=jax.ShapeDtypeStruct((BATCH, EMBED), jnp.float32),
        mesh=mesh,
        compiler_params=pltpu.CompilerParams(use_tc_tiling_on_sc=False,
                                             needs_layout_passes=False),
        scratch_types=[
            pltpu.VMEM((U_W, IPU), jnp.int32),
            pltpu.VMEM((NBUF, IPU, EMBED), jnp.float32),
            pltpu.VMEM((B_W, EMBED), jnp.float32),
        ] + [pltpu.SemaphoreType.DMA] * NBUF,
    )(idx, table_lin)


def kernel(input_var, table):
    idx = input_var.astype(jnp.int32).reshape(BATCH // SPU, IPU)
    return _emb_mean(idx, table)
